# baseline (speedup 1.0000x reference)
import functools

import jax
import jax.numpy as jnp
from jax import lax
from jax.experimental import pallas as pl
from jax.experimental.pallas import tpu as pltpu
from jax.experimental.pallas import tpu_sc as plsc

N, WAY, SHOT, C, H, W = 4, 5, 5, 256, 32, 32
NUM_CLASSES = 1024
B = N * WAY * SHOT
HW = H * W
BB = 4
NK = N * WAY
NKP = 32
TOT_ROWS = N * NUM_CLASSES
DUMP = TOT_ROWS
NWORK = 32
ROWS_PER_W = TOT_ROWS // NWORK
L = 16

_HIGH = lax.Precision.HIGHEST


def _k1_body(f_ref, m_ref, s_ref):
    for i in range(BB):
        s_ref[i] = lax.dot_general(
            f_ref[i], m_ref[i],
            dimension_numbers=(((1,), (0,)), ((), ())),
            preferred_element_type=jnp.float32,
        )


def _k2_body(s_ref, m_ref, w1_ref, w2_ref, wm_ref, b1_ref, b2_ref, bm_ref,
             keyt_ref, keyj_ref, proto_ref):
    S = s_ref[...]
    att = S * (1.0 / HW)
    h = lax.dot_general(att, w1_ref[...], (((1,), (1,)), ((), ())),
                        preferred_element_type=jnp.float32, precision=_HIGH)
    h = jnp.maximum(h + b1_ref[...], 0.0)
    h = lax.dot_general(h, w2_ref[...], (((1,), (1,)), ((), ())),
                        preferred_element_type=jnp.float32, precision=_HIGH)
    g = jax.nn.sigmoid(h + b2_ref[...])
    denom = jnp.sum(m_ref[...], axis=2)
    proto = (g * S) / (denom + 1e-3)
    proto = lax.dot_general(proto, wm_ref[...], (((1,), (1,)), ((), ())),
                            preferred_element_type=jnp.float32,
                            precision=_HIGH)
    proto = proto + bm_ref[...]
    r = lax.broadcasted_iota(jnp.int32, (NK, B), 0)
    c = lax.broadcasted_iota(jnp.int32, (NK, B), 1)
    sel_mean = jnp.where((c >= r * SHOT) & (c < r * SHOT + SHOT),
                         1.0 / SHOT, 0.0)
    pm = lax.dot_general(sel_mean, proto, (((1,), (0,)), ((), ())),
                         preferred_element_type=jnp.float32, precision=_HIGH)
    j = lax.broadcasted_iota(jnp.int32, (NKP, NK), 1)
    match = keyt_ref[...] == keyj_ref[...]
    lastm = jnp.max(jnp.where(match, j, -1), axis=1, keepdims=True)
    sel_last = jnp.where(j == lastm, 1.0, 0.0)
    proto_ref[...] = lax.dot_general(sel_last, pm, (((1,), (0,)), ((), ())),
                                     preferred_element_type=jnp.float32,
                                     precision=_HIGH)


def _ka_body(weight_hbm, out_hbm, chunk_v):
    cid = lax.axis_index("c")
    sid = lax.axis_index("s")
    wid = sid * 2 + cid
    lo = wid * ROWS_PER_W
    wlo = lax.rem(lo, NUM_CLASSES)
    pltpu.sync_copy(weight_hbm.at[pl.ds(wlo, ROWS_PER_W)], chunk_v)
    pltpu.sync_copy(chunk_v, out_hbm.at[pl.ds(lo, ROWS_PER_W)])


@functools.cache
def _ka_broadcast():
    return pl.kernel(
        _ka_body,
        out_type=jax.ShapeDtypeStruct((TOT_ROWS, C), jnp.float32),
        mesh=plsc.VectorSubcoreMesh(core_axis_name="c", subcore_axis_name="s"),
        scratch_types=[
            pltpu.VMEM((ROWS_PER_W, C), jnp.float32),
        ],
        compiler_params=pltpu.CompilerParams(use_tc_tiling_on_sc=True),
    )


def _k4_body(lab_ref, outin_ref, proto_ref, out_ref, sem):
    del outin_ref
    copies = []
    for k in range(NK):
        row = lab_ref[k]
        copies.append(pltpu.make_async_copy(
            proto_ref.at[pl.ds(k, 1)], out_ref.at[pl.ds(row, 1)], sem))
    for cp in copies:
        cp.start()
    for cp in copies:
        cp.wait()


def _k4_scatter(key, outb, proto):
    return pl.pallas_call(
        _k4_body,
        grid_spec=pltpu.PrefetchScalarGridSpec(
            num_scalar_prefetch=1,
            grid=(1,),
            in_specs=[
                pl.BlockSpec(memory_space=pl.ANY),
                pl.BlockSpec((NKP, C), lambda i, s: (0, 0)),
            ],
            out_specs=pl.BlockSpec(memory_space=pl.ANY),
            scratch_shapes=[pltpu.SemaphoreType.DMA],
        ),
        out_shape=jax.ShapeDtypeStruct((TOT_ROWS, C), jnp.float32),
        input_output_aliases={1: 0},
    )(key, outb, proto)


def kernel(Fs, Ys, labels, W1, b1, W2, b2, Wm, bm, weight):
    f3 = Fs.astype(jnp.bfloat16).reshape(B, C, HW)
    m5 = Ys.reshape(B, 1, HW).astype(jnp.float32)
    m5h = m5.reshape(B, HW, 1).astype(jnp.bfloat16)

    s3 = pl.pallas_call(
        _k1_body,
        grid=(B // BB,),
        in_specs=[
            pl.BlockSpec((BB, C, HW), lambda i: (i, 0, 0)),
            pl.BlockSpec((BB, HW, 1), lambda i: (i, 0, 0)),
        ],
        out_specs=pl.BlockSpec((BB, C, 1), lambda i: (i, 0, 0)),
        out_shape=jax.ShapeDtypeStruct((B, C, 1), jnp.float32),
    )(f3, m5h)
    s3 = s3.reshape(B, C)

    key = (labels + NUM_CLASSES * lax.broadcasted_iota(jnp.int32, (N, WAY), 0)
           ).reshape(NK)
    keyp = jnp.pad(key, (0, NKP - NK), constant_values=-1)
    keyt = jnp.broadcast_to(keyp[:, None], (NKP, NK))
    keyj = jnp.broadcast_to(key[None, :], (NKP, NK))

    proto = pl.pallas_call(
        _k2_body,
        out_shape=jax.ShapeDtypeStruct((NKP, C), jnp.float32),
    )(s3, m5, W1, W2, Wm,
      b1.reshape(1, C), b2.reshape(1, C), bm.reshape(1, C), keyt, keyj)

    outb = _ka_broadcast()(weight)
    out = _k4_scatter(key, outb, proto)
    return out.reshape(N, NUM_CLASSES, C)


# speedup vs baseline: 1.5766x; 1.5766x over previous
import functools

import jax
import jax.numpy as jnp
from jax import lax
from jax.experimental import pallas as pl
from jax.experimental.pallas import tpu as pltpu
from jax.experimental.pallas import tpu_sc as plsc

N, WAY, SHOT, C, H, W = 4, 5, 5, 256, 32, 32
NUM_CLASSES = 1024
B = N * WAY * SHOT
HW = H * W
BB = 4
NK = N * WAY
NKP = 32
TOT_ROWS = N * NUM_CLASSES
DUMP = TOT_ROWS
NWORK = 32
ROWS_PER_W = TOT_ROWS // NWORK
L = 16

_HIGH = lax.Precision.HIGHEST


def _k1_body(f_ref, m_ref, s_ref):
    s_ref[0] = jnp.sum(f_ref[...] * m_ref[...], axis=2)


def _k2_body(s_ref, m_ref, w1_ref, w2_ref, wm_ref, b1_ref, b2_ref, bm_ref,
             keyt_ref, keyj_ref, proto_ref):
    S = s_ref[...]
    att = S * (1.0 / HW)
    h = lax.dot_general(att, w1_ref[...], (((1,), (1,)), ((), ())),
                        preferred_element_type=jnp.float32, precision=_HIGH)
    h = jnp.maximum(h + b1_ref[...], 0.0)
    h = lax.dot_general(h, w2_ref[...], (((1,), (1,)), ((), ())),
                        preferred_element_type=jnp.float32, precision=_HIGH)
    g = jax.nn.sigmoid(h + b2_ref[...])
    denom = jnp.sum(m_ref[...], axis=2)
    proto = (g * S) / (denom + 1e-3)
    proto = lax.dot_general(proto, wm_ref[...], (((1,), (1,)), ((), ())),
                            preferred_element_type=jnp.float32,
                            precision=_HIGH)
    proto = proto + bm_ref[...]
    r = lax.broadcasted_iota(jnp.int32, (NK, B), 0)
    c = lax.broadcasted_iota(jnp.int32, (NK, B), 1)
    sel_mean = jnp.where((c >= r * SHOT) & (c < r * SHOT + SHOT),
                         1.0 / SHOT, 0.0)
    pm = lax.dot_general(sel_mean, proto, (((1,), (0,)), ((), ())),
                         preferred_element_type=jnp.float32, precision=_HIGH)
    j = lax.broadcasted_iota(jnp.int32, (NKP, NK), 1)
    match = keyt_ref[...] == keyj_ref[...]
    lastm = jnp.max(jnp.where(match, j, -1), axis=1, keepdims=True)
    sel_last = jnp.where(j == lastm, 1.0, 0.0)
    proto_ref[...] = lax.dot_general(sel_last, pm, (((1,), (0,)), ((), ())),
                                     preferred_element_type=jnp.float32,
                                     precision=_HIGH)


def _ka_body(weight_hbm, out_hbm, chunk_v):
    cid = lax.axis_index("c")
    sid = lax.axis_index("s")
    wid = sid * 2 + cid
    lo = wid * ROWS_PER_W
    wlo = lax.rem(lo, NUM_CLASSES)
    pltpu.sync_copy(weight_hbm.at[pl.ds(wlo, ROWS_PER_W)], chunk_v)
    pltpu.sync_copy(chunk_v, out_hbm.at[pl.ds(lo, ROWS_PER_W)])


@functools.cache
def _ka_broadcast():
    return pl.kernel(
        _ka_body,
        out_type=jax.ShapeDtypeStruct((TOT_ROWS, C), jnp.float32),
        mesh=plsc.VectorSubcoreMesh(core_axis_name="c", subcore_axis_name="s"),
        scratch_types=[
            pltpu.VMEM((ROWS_PER_W, C), jnp.float32),
        ],
        compiler_params=pltpu.CompilerParams(use_tc_tiling_on_sc=True),
    )


def _k4_body(lab_ref, outin_ref, s_ref, m_ref, w1_ref, w2_ref, wm_ref,
             b1_ref, b2_ref, bm_ref, keyt_ref, keyj_ref, out_ref,
             proto_scr, sem):
    del outin_ref
    _k2_body(s_ref, m_ref, w1_ref, w2_ref, wm_ref, b1_ref, b2_ref, bm_ref,
             keyt_ref, keyj_ref, proto_scr)
    copies = []
    for k in range(NK):
        row = lab_ref[k]
        copies.append(pltpu.make_async_copy(
            proto_scr.at[pl.ds(k, 1)], out_ref.at[pl.ds(row, 1)], sem))
    for cp in copies:
        cp.start()
    for cp in copies:
        cp.wait()


def _k4_scatter(key, outb, s3, m5, W1, W2, Wm, b1r, b2r, bmr, keyt, keyj):
    full = lambda x: pl.BlockSpec(x.shape, lambda i, s, n=x.ndim: (0,) * n)
    return pl.pallas_call(
        _k4_body,
        grid_spec=pltpu.PrefetchScalarGridSpec(
            num_scalar_prefetch=1,
            grid=(1,),
            in_specs=[
                pl.BlockSpec(memory_space=pl.ANY),
                full(s3), full(m5), full(W1), full(W2), full(Wm),
                full(b1r), full(b2r), full(bmr), full(keyt), full(keyj),
            ],
            out_specs=pl.BlockSpec(memory_space=pl.ANY),
            scratch_shapes=[pltpu.VMEM((NKP, C), jnp.float32),
                            pltpu.SemaphoreType.DMA],
        ),
        out_shape=jax.ShapeDtypeStruct((TOT_ROWS, C), jnp.float32),
        input_output_aliases={1: 0},
    )(key, outb, s3, m5, W1, W2, Wm, b1r, b2r, bmr, keyt, keyj)


def kernel(Fs, Ys, labels, W1, b1, W2, b2, Wm, bm, weight):
    f3 = Fs.reshape(B, C, HW)
    m5 = Ys.reshape(B, 1, HW).astype(jnp.float32)

    s3 = pl.pallas_call(
        _k1_body,
        grid=(B // BB,),
        in_specs=[
            pl.BlockSpec((BB, C, HW), lambda i: (i, 0, 0)),
            pl.BlockSpec((BB, 1, HW), lambda i: (i, 0, 0)),
        ],
        out_specs=pl.BlockSpec((1, BB, C), lambda i: (i, 0, 0)),
        out_shape=jax.ShapeDtypeStruct((B // BB, BB, C), jnp.float32),
    )(f3, m5)
    s3 = s3.reshape(B, C)

    key = (labels + NUM_CLASSES * lax.broadcasted_iota(jnp.int32, (N, WAY), 0)
           ).reshape(NK)
    keyp = jnp.pad(key, (0, NKP - NK), constant_values=-1)
    keyt = jnp.broadcast_to(keyp[:, None], (NKP, NK))
    keyj = jnp.broadcast_to(key[None, :], (NKP, NK))

    outb = _ka_broadcast()(weight)
    out = _k4_scatter(key, outb, s3, m5, W1, W2, Wm,
                      b1.reshape(1, C), b2.reshape(1, C), bm.reshape(1, C),
                      keyt, keyj)
    return out.reshape(N, NUM_CLASSES, C)


# speedup vs baseline: 1.7329x; 1.0991x over previous
import functools

import jax
import jax.numpy as jnp
from jax import lax
from jax.experimental import pallas as pl
from jax.experimental.pallas import tpu as pltpu
from jax.experimental.pallas import tpu_sc as plsc

N, WAY, SHOT, C, H, W = 4, 5, 5, 256, 32, 32
NUM_CLASSES = 1024
B = N * WAY * SHOT
HW = H * W
BB = 10
NK = N * WAY
NKP = 32
TOT_ROWS = N * NUM_CLASSES
DUMP = TOT_ROWS
NWORK = 32
ROWS_PER_W = TOT_ROWS // NWORK
L = 16

_HIGH = lax.Precision.HIGHEST


def _k1_body(f_ref, m_ref, s_ref):
    s_ref[0] = jnp.sum(f_ref[...] * m_ref[...], axis=2)


def _k2_body(s_ref, m_ref, w1_ref, w2_ref, wm_ref, b1_ref, b2_ref, bm_ref,
             keyt_ref, keyj_ref, proto_ref):
    S = s_ref[...]
    att = S * (1.0 / HW)
    h = lax.dot_general(att, w1_ref[...], (((1,), (1,)), ((), ())),
                        preferred_element_type=jnp.float32, precision=_HIGH)
    h = jnp.maximum(h + b1_ref[...], 0.0)
    h = lax.dot_general(h, w2_ref[...], (((1,), (1,)), ((), ())),
                        preferred_element_type=jnp.float32, precision=_HIGH)
    g = jax.nn.sigmoid(h + b2_ref[...])
    denom = jnp.sum(m_ref[...], axis=2)
    proto = (g * S) / (denom + 1e-3)
    proto = lax.dot_general(proto, wm_ref[...], (((1,), (1,)), ((), ())),
                            preferred_element_type=jnp.float32,
                            precision=_HIGH)
    proto = proto + bm_ref[...]
    r = lax.broadcasted_iota(jnp.int32, (NK, B), 0)
    c = lax.broadcasted_iota(jnp.int32, (NK, B), 1)
    sel_mean = jnp.where((c >= r * SHOT) & (c < r * SHOT + SHOT),
                         1.0 / SHOT, 0.0)
    pm = lax.dot_general(sel_mean, proto, (((1,), (0,)), ((), ())),
                         preferred_element_type=jnp.float32, precision=_HIGH)
    j = lax.broadcasted_iota(jnp.int32, (NKP, NK), 1)
    match = keyt_ref[...] == keyj_ref[...]
    lastm = jnp.max(jnp.where(match, j, -1), axis=1, keepdims=True)
    sel_last = jnp.where(j == lastm, 1.0, 0.0)
    proto_ref[...] = lax.dot_general(sel_last, pm, (((1,), (0,)), ((), ())),
                                     preferred_element_type=jnp.float32,
                                     precision=_HIGH)


def _ka_body(weight_hbm, out_hbm, chunk_v):
    cid = lax.axis_index("c")
    sid = lax.axis_index("s")
    wid = sid * 2 + cid
    lo = wid * ROWS_PER_W
    wlo = lax.rem(lo, NUM_CLASSES)
    pltpu.sync_copy(weight_hbm.at[pl.ds(wlo, ROWS_PER_W)], chunk_v)
    pltpu.sync_copy(chunk_v, out_hbm.at[pl.ds(lo, ROWS_PER_W)])


@functools.cache
def _ka_broadcast():
    return pl.kernel(
        _ka_body,
        out_type=jax.ShapeDtypeStruct((TOT_ROWS, C), jnp.float32),
        mesh=plsc.VectorSubcoreMesh(core_axis_name="c", subcore_axis_name="s"),
        scratch_types=[
            pltpu.VMEM((ROWS_PER_W, C), jnp.float32),
        ],
        compiler_params=pltpu.CompilerParams(use_tc_tiling_on_sc=True),
    )


def _k4_body(lab_ref, outin_ref, s_ref, m_ref, w1_ref, w2_ref, wm_ref,
             b1_ref, b2_ref, bm_ref, keyt_ref, keyj_ref, out_ref,
             proto_scr, sem):
    del outin_ref
    _k2_body(s_ref, m_ref, w1_ref, w2_ref, wm_ref, b1_ref, b2_ref, bm_ref,
             keyt_ref, keyj_ref, proto_scr)
    copies = []
    for k in range(NK):
        row = lab_ref[k]
        copies.append(pltpu.make_async_copy(
            proto_scr.at[pl.ds(k, 1)], out_ref.at[pl.ds(row, 1)], sem))
    for cp in copies:
        cp.start()
    for cp in copies:
        cp.wait()


def _k4_scatter(key, outb, s3, m5, W1, W2, Wm, b1r, b2r, bmr, keyt, keyj):
    full = lambda x: pl.BlockSpec(x.shape, lambda i, s, n=x.ndim: (0,) * n)
    return pl.pallas_call(
        _k4_body,
        grid_spec=pltpu.PrefetchScalarGridSpec(
            num_scalar_prefetch=1,
            grid=(1,),
            in_specs=[
                pl.BlockSpec(memory_space=pl.ANY),
                full(s3), full(m5), full(W1), full(W2), full(Wm),
                full(b1r), full(b2r), full(bmr), full(keyt), full(keyj),
            ],
            out_specs=pl.BlockSpec(memory_space=pl.ANY),
            scratch_shapes=[pltpu.VMEM((NKP, C), jnp.float32),
                            pltpu.SemaphoreType.DMA],
        ),
        out_shape=jax.ShapeDtypeStruct((TOT_ROWS, C), jnp.float32),
        input_output_aliases={1: 0},
    )(key, outb, s3, m5, W1, W2, Wm, b1r, b2r, bmr, keyt, keyj)


def kernel(Fs, Ys, labels, W1, b1, W2, b2, Wm, bm, weight):
    f3 = Fs.reshape(B, C, HW)
    m5 = Ys.reshape(B, 1, HW).astype(jnp.float32)

    s3 = pl.pallas_call(
        _k1_body,
        grid=(B // BB,),
        in_specs=[
            pl.BlockSpec((BB, C, HW), lambda i: (i, 0, 0)),
            pl.BlockSpec((BB, 1, HW), lambda i: (i, 0, 0)),
        ],
        out_specs=pl.BlockSpec((1, BB, C), lambda i: (i, 0, 0)),
        out_shape=jax.ShapeDtypeStruct((B // BB, BB, C), jnp.float32),
    )(f3, m5)
    s3 = s3.reshape(B, C)

    key = (labels + NUM_CLASSES * lax.broadcasted_iota(jnp.int32, (N, WAY), 0)
           ).reshape(NK)
    keyp = jnp.pad(key, (0, NKP - NK), constant_values=-1)
    keyt = jnp.broadcast_to(keyp[:, None], (NKP, NK))
    keyj = jnp.broadcast_to(key[None, :], (NKP, NK))

    outb = _ka_broadcast()(weight)
    out = _k4_scatter(key, outb, s3, m5, W1, W2, Wm,
                      b1.reshape(1, C), b2.reshape(1, C), bm.reshape(1, C),
                      keyt, keyj)
    return out.reshape(N, NUM_CLASSES, C)
